# baseline (speedup 1.0000x reference)
import jax
import jax.numpy as jnp
from jax import lax
from jax.experimental import pallas as pl
from jax.experimental.pallas import tpu as pltpu
from jax.experimental.pallas import tpu_sc as plsc

D = 32
H = 768
NC = 2
NS = 16
NW = NC * NS
CH = 128
SPG = 2
GROUP = CH * SPG
OROWS = GROUP * D // CH
PACK = CH // D


def _repack_tc(t):
  v = t.shape[0]
  bm = 8000

  def body(x_ref, o_ref):
    x3 = x_ref[...].reshape(bm // PACK, PACK, D)
    o_ref[...] = jnp.concatenate([x3[:, j, :] for j in range(PACK)], axis=1)

  return pl.pallas_call(
      body, grid=(v // bm,),
      in_specs=[pl.BlockSpec((bm, D), lambda i: (i, 0))],
      out_specs=pl.BlockSpec((bm // PACK, PACK * D), lambda i: (i, 0)),
      out_shape=jax.ShapeDtypeStruct((v // PACK, PACK * D), jnp.float32),
  )(t)


def _gather_pipe(ids_ref, tab_ref, out_ref, n, wid,
                 ids_v, sidx, cbase, gbuf, obuf, sems):
  per_w = n // NW
  groups = per_w // GROUP
  idrow0 = wid * (per_w // CH)
  orow0 = wid * (per_w * D // CH)

  def load_and_fire(g, p):
    pltpu.sync_copy(ids_ref.at[pl.ds(idrow0 + g * SPG, SPG)],
                    ids_v.at[pl.ds(p * SPG, SPG)])
    for j in range(SPG):
      row = ids_v.at[p * SPG + j]
      for t in range(CH // 16):
        raw = row[pl.ds(t * 16, 16)]
        base = p * GROUP + j * CH + t * 16
        sidx[pl.ds(base, 16)] = lax.shift_right_logical(raw, 2)
        cbase[pl.ds(base, 16)] = (raw & (PACK - 1)) * D
    return [pltpu.async_copy(tab_ref.at[sidx.at[pl.ds(p * GROUP + j * CH, CH)]],
                             gbuf.at[pl.ds(p * GROUP + j * CH, CH)], sems[p])
            for j in range(SPG)]

  def drain_extract(g, p, cps):
    for cp in cps:
      cp.wait()

    @pl.loop(0, GROUP // 16)
    def _t(t):
      r0 = pl.multiple_of(t * 16, 16)
      rows = r0 + jnp.arange(16, dtype=jnp.int32)
      cb = cbase[pl.ds(p * GROUP + r0, 16)]
      grows = rows + p * GROUP
      pbase = rows * D
      for c in range(D):
        v = plsc.load_gather(gbuf, [grows, cb + c])
        pp = pbase + c
        plsc.store_scatter(obuf, [lax.shift_right_logical(pp, 7), pp & 127], v)

    pltpu.sync_copy(obuf, out_ref.at[pl.ds(orow0 + g * OROWS, OROWS)])

  cps0 = load_and_fire(0, 0)

  @pl.loop(0, groups // 2)
  def _k(k):
    g0 = k * 2
    cps1 = load_and_fire(g0 + 1, 1)
    drain_extract(g0, 0, cps0)

    @pl.when(k < groups // 2 - 1)
    def _pre():
      load_and_fire(g0 + 2, 0)

    drain_extract(g0 + 1, 1, cps1)


def _sc_gather(item_idx, user_idx, item_tab, user_tab, ni, nu):
  mesh = plsc.VectorSubcoreMesh(core_axis_name="c", subcore_axis_name="s")

  def body(item_idx_ref, user_idx_ref, item_tab_ref, user_tab_ref,
           items_out, users_out, ids_v, sidx, cbase, gbuf, obuf,
           sem0, sem1):
    wid = lax.axis_index("s") * NC + lax.axis_index("c")
    _gather_pipe(item_idx_ref, item_tab_ref, items_out, ni, wid,
                 ids_v, sidx, cbase, gbuf, obuf, (sem0, sem1))
    _gather_pipe(user_idx_ref, user_tab_ref, users_out, nu, wid,
                 ids_v, sidx, cbase, gbuf, obuf, (sem0, sem1))

  fn = pl.kernel(
      body,
      out_type=(jax.ShapeDtypeStruct((ni * D // CH, CH), jnp.float32),
                jax.ShapeDtypeStruct((nu * D // CH, CH), jnp.float32)),
      mesh=mesh,
      compiler_params=pltpu.CompilerParams(use_tc_tiling_on_sc=True,
                                           needs_layout_passes=False),
      scratch_types=[
          pltpu.VMEM((2 * SPG, CH), jnp.int32),
          pltpu.VMEM((2 * GROUP,), jnp.int32),
          pltpu.VMEM((2 * GROUP,), jnp.int32),
          pltpu.VMEM((2 * GROUP, CH), jnp.float32),
          pltpu.VMEM((OROWS, CH), jnp.float32),
          pltpu.SemaphoreType.DMA,
          pltpu.SemaphoreType.DMA,
      ],
  )
  return fn(item_idx, user_idx, item_tab, user_tab)


def _project_packed(x4, w, bm4):
  m4 = x4.shape[0]

  def mm(x_ref, w_ref, o_ref):
    x = x_ref[...]
    for o in range(PACK):
      o_ref[:, o * H:(o + 1) * H] = lax.dot_general(
          x[:, o * D:(o + 1) * D], w_ref[...],
          (((1,), (1,)), ((), ())), preferred_element_type=jnp.float32)

  return pl.pallas_call(
      mm,
      grid=(m4 // bm4,),
      in_specs=[pl.BlockSpec((bm4, PACK * D), lambda i: (i, 0)),
                pl.BlockSpec((H, D), lambda i: (0, 0))],
      out_specs=pl.BlockSpec((bm4, PACK * H), lambda i: (i, 0)),
      out_shape=jax.ShapeDtypeStruct((m4, PACK * H), jnp.float32),
  )(x4, w)


def kernel(user_ids, item_ids, user_table, item_table, W_user, W_item):
  b, l = item_ids.shape
  ni = b * l
  item_idx = item_ids.reshape(ni // CH, CH)
  user_idx = user_ids.reshape(b // CH, CH)
  t_item = _repack_tc(item_table)
  t_user = _repack_tc(user_table)
  items_f, users_f = _sc_gather(item_idx, user_idx, t_item, t_user, ni, b)
  u_proj = _project_packed(users_f, W_user, 512).reshape(b, H)
  i_proj = _project_packed(items_f, W_item, 512).reshape(b, l, H)
  return (u_proj, i_proj)


# speedup vs baseline: 1.2691x; 1.2691x over previous
import jax
import jax.numpy as jnp
from jax import lax
from jax.experimental import pallas as pl
from jax.experimental.pallas import tpu as pltpu
from jax.experimental.pallas import tpu_sc as plsc

D = 32
H = 768
NC = 2
NS = 16
NW = NC * NS
CH = 128
SPG = 2
GROUP = CH * SPG
OROWS = GROUP * D // CH
PACK = CH // D


def _repack_tc(t):
  tt = jnp.transpose(t)
  v = t.shape[0]
  bn = 8192

  def body(x_ref, o_ref):
    xt = x_ref[...].T
    x3 = xt.reshape(bn // PACK, PACK, D)
    o_ref[...] = jnp.concatenate([x3[:, j, :] for j in range(PACK)], axis=1)

  return pl.pallas_call(
      body, grid=(pl.cdiv(v, bn),),
      in_specs=[pl.BlockSpec((D, bn), lambda i: (0, i))],
      out_specs=pl.BlockSpec((bn // PACK, PACK * D), lambda i: (i, 0)),
      out_shape=jax.ShapeDtypeStruct((v // PACK, PACK * D), jnp.float32),
  )(tt)


def _gather_pipe(ids_ref, tab_ref, out_ref, n, wid,
                 ids_v, sidx, cbase, gbuf, obuf, sems):
  per_w = n // NW
  groups = per_w // GROUP
  idrow0 = wid * (per_w // CH)
  orow0 = wid * (per_w * D // CH)

  def load_and_fire(g, p):
    pltpu.sync_copy(ids_ref.at[pl.ds(idrow0 + g * SPG, SPG)],
                    ids_v.at[pl.ds(p * SPG, SPG)])
    for j in range(SPG):
      row = ids_v.at[p * SPG + j]
      for t in range(CH // 16):
        raw = row[pl.ds(t * 16, 16)]
        base = p * GROUP + j * CH + t * 16
        sidx[pl.ds(base, 16)] = lax.shift_right_logical(raw, 2)
        cbase[pl.ds(base, 16)] = (raw & (PACK - 1)) * D
    return [pltpu.async_copy(tab_ref.at[sidx.at[pl.ds(p * GROUP + j * CH, CH)]],
                             gbuf.at[pl.ds(p * GROUP + j * CH, CH)], sems[p])
            for j in range(SPG)]

  def drain_extract(g, p, cps):
    for cp in cps:
      cp.wait()

    @pl.loop(0, GROUP // 16)
    def _t(t):
      r0 = pl.multiple_of(t * 16, 16)
      rows = r0 + jnp.arange(16, dtype=jnp.int32)
      cb = cbase[pl.ds(p * GROUP + r0, 16)]
      grows = rows + p * GROUP
      pbase = rows * D
      for c in range(D):
        v = plsc.load_gather(gbuf, [grows, cb + c])
        pp = pbase + c
        plsc.store_scatter(obuf, [lax.shift_right_logical(pp, 7), pp & 127], v)

    pltpu.sync_copy(obuf, out_ref.at[pl.ds(orow0 + g * OROWS, OROWS)])

  cps0 = load_and_fire(0, 0)

  @pl.loop(0, groups // 2)
  def _k(k):
    g0 = k * 2
    cps1 = load_and_fire(g0 + 1, 1)
    drain_extract(g0, 0, cps0)

    @pl.when(k < groups // 2 - 1)
    def _pre():
      load_and_fire(g0 + 2, 0)

    drain_extract(g0 + 1, 1, cps1)


def _sc_gather(item_idx, user_idx, item_tab, user_tab, ni, nu):
  mesh = plsc.VectorSubcoreMesh(core_axis_name="c", subcore_axis_name="s")

  def body(item_idx_ref, user_idx_ref, item_tab_ref, user_tab_ref,
           items_out, users_out, ids_v, sidx, cbase, gbuf, obuf,
           sem0, sem1):
    wid = lax.axis_index("s") * NC + lax.axis_index("c")
    _gather_pipe(item_idx_ref, item_tab_ref, items_out, ni, wid,
                 ids_v, sidx, cbase, gbuf, obuf, (sem0, sem1))
    _gather_pipe(user_idx_ref, user_tab_ref, users_out, nu, wid,
                 ids_v, sidx, cbase, gbuf, obuf, (sem0, sem1))

  fn = pl.kernel(
      body,
      out_type=(jax.ShapeDtypeStruct((ni * D // CH, CH), jnp.float32),
                jax.ShapeDtypeStruct((nu * D // CH, CH), jnp.float32)),
      mesh=mesh,
      compiler_params=pltpu.CompilerParams(use_tc_tiling_on_sc=True,
                                           needs_layout_passes=False),
      scratch_types=[
          pltpu.VMEM((2 * SPG, CH), jnp.int32),
          pltpu.VMEM((2 * GROUP,), jnp.int32),
          pltpu.VMEM((2 * GROUP,), jnp.int32),
          pltpu.VMEM((2 * GROUP, CH), jnp.float32),
          pltpu.VMEM((OROWS, CH), jnp.float32),
          pltpu.SemaphoreType.DMA,
          pltpu.SemaphoreType.DMA,
      ],
  )
  return fn(item_idx, user_idx, item_tab, user_tab)


def _project_packed(x4, w, bm4):
  m4 = x4.shape[0]

  def mm(x_ref, w_ref, o_ref):
    x = x_ref[...]
    for o in range(PACK):
      o_ref[:, o * H:(o + 1) * H] = lax.dot_general(
          x[:, o * D:(o + 1) * D], w_ref[...],
          (((1,), (1,)), ((), ())), preferred_element_type=jnp.float32)

  return pl.pallas_call(
      mm,
      grid=(m4 // bm4,),
      in_specs=[pl.BlockSpec((bm4, PACK * D), lambda i: (i, 0)),
                pl.BlockSpec((H, D), lambda i: (0, 0))],
      out_specs=pl.BlockSpec((bm4, PACK * H), lambda i: (i, 0)),
      out_shape=jax.ShapeDtypeStruct((m4, PACK * H), jnp.float32),
  )(x4, w)


def kernel(user_ids, item_ids, user_table, item_table, W_user, W_item):
  b, l = item_ids.shape
  ni = b * l
  item_idx = item_ids.T.reshape(ni // CH, CH)
  user_idx = user_ids.reshape(b // CH, CH)
  t_item = _repack_tc(item_table)
  t_user = _repack_tc(user_table)
  items_f, users_f = _sc_gather(item_idx, user_idx, t_item, t_user, ni, b)
  u_proj = _project_packed(users_f, W_user, 512).reshape(b, H)
  i_proj = _project_packed(items_f, W_item, 512).reshape(l, b, H)
  return (u_proj, i_proj.transpose(1, 0, 2))


# speedup vs baseline: 1.2892x; 1.0158x over previous
import jax
import jax.numpy as jnp
from jax import lax
from jax.experimental import pallas as pl
from jax.experimental.pallas import tpu as pltpu
from jax.experimental.pallas import tpu_sc as plsc

D = 32
H = 768
NC = 2
NS = 16
NW = NC * NS
CH = 128
SPG = 2
GROUP = CH * SPG
OROWS = GROUP * D // CH
PACK = CH // D


def _repack_tc(t):
  tt = jnp.transpose(t)
  v = t.shape[0]
  bn = 8192

  def body(x_ref, o_ref):
    xt = x_ref[...].T
    x3 = xt.reshape(bn // PACK, PACK, D)
    o_ref[...] = jnp.concatenate([x3[:, j, :] for j in range(PACK)], axis=1)

  return pl.pallas_call(
      body, grid=(pl.cdiv(v, bn),),
      in_specs=[pl.BlockSpec((D, bn), lambda i: (0, i))],
      out_specs=pl.BlockSpec((bn // PACK, PACK * D), lambda i: (i, 0)),
      out_shape=jax.ShapeDtypeStruct((v // PACK, PACK * D), jnp.float32),
  )(tt)


def _gather_pipe(ids_ref, tab_ref, out_ref, n, wid,
                 ids_v, sidx, cbase, gbuf, obuf,
                 sem_i, sem_g, sem_o):
  per_w = n // NW
  groups = per_w // GROUP
  idrow0 = wid * (per_w // CH)
  orow0 = wid * (per_w * D // CH)

  def fire_ids(g, p):
    return pltpu.async_copy(ids_ref.at[pl.ds(idrow0 + g * SPG, SPG)],
                            ids_v.at[pl.ds(p * SPG, SPG)], sem_i)

  def idx_compute(p):
    for j in range(SPG):
      row = ids_v.at[p * SPG + j]
      for t in range(CH // 16):
        raw = row[pl.ds(t * 16, 16)]
        base = p * GROUP + j * CH + t * 16
        sidx[pl.ds(base, 16)] = lax.shift_right_logical(raw, 2)
        cbase[pl.ds(base, 16)] = (raw & (PACK - 1)) * D

  def fire_gather(p):
    return [pltpu.async_copy(
        tab_ref.at[sidx.at[pl.ds(p * GROUP + j * CH, CH)]],
        gbuf.at[pl.ds(p * GROUP + j * CH, CH)], sem_g)
        for j in range(SPG)]

  def fire_out(g):
    return pltpu.async_copy(obuf, out_ref.at[pl.ds(orow0 + g * OROWS, OROWS)],
                            sem_o)

  def extract(p):
    @pl.loop(0, GROUP // 16)
    def _t(t):
      r0 = pl.multiple_of(t * 16, 16)
      rows = r0 + jnp.arange(16, dtype=jnp.int32)
      cb = cbase[pl.ds(p * GROUP + r0, 16)]
      grows = rows + p * GROUP
      pbase = rows * D
      for c in range(D):
        v = plsc.load_gather(gbuf, [grows, cb + c])
        pp = pbase + c
        plsc.store_scatter(obuf, [lax.shift_right_logical(pp, 7), pp & 127], v)

  ids_t = fire_ids(0, 0)
  ids_t.wait()
  idx_compute(0)
  gat_t = fire_gather(0)
  out_t = None
  if groups > 1:
    ids_t2 = fire_ids(1, 1)

  def group_body(g, p):
    q = 1 - p

    @pl.when(g < groups - 1)
    def _prep():
      ids_t.wait()
      idx_compute(q)

      @pl.when(g < groups - 2)
      def _pre_ids():
        fire_ids(g + 2, p)

      fire_gather(q)

    for cp in gat_t:
      cp.wait()

    @pl.when(g >= 1)
    def _wout():
      nonlocal_out_wait()

    extract(p)
    fire_out(g)

  def nonlocal_out_wait():
    pltpu.make_async_copy(obuf, out_ref.at[pl.ds(orow0, OROWS)],
                          sem_o).wait()

  @pl.loop(0, groups // 2)
  def _k(k):
    group_body(2 * k, 0)
    group_body(2 * k + 1, 1)

  nonlocal_out_wait()


def _sc_gather(item_idx, user_idx, item_tab, user_tab, ni, nu):
  mesh = plsc.VectorSubcoreMesh(core_axis_name="c", subcore_axis_name="s")

  def body(item_idx_ref, user_idx_ref, item_tab_ref, user_tab_ref,
           items_out, users_out, ids_v, sidx, cbase, gbuf, obuf,
           sem_i, sem_g, sem_o):
    wid = lax.axis_index("s") * NC + lax.axis_index("c")
    _gather_pipe(item_idx_ref, item_tab_ref, items_out, ni, wid,
                 ids_v, sidx, cbase, gbuf, obuf, sem_i, sem_g, sem_o)
    _gather_pipe(user_idx_ref, user_tab_ref, users_out, nu, wid,
                 ids_v, sidx, cbase, gbuf, obuf, sem_i, sem_g, sem_o)

  fn = pl.kernel(
      body,
      out_type=(jax.ShapeDtypeStruct((ni * D // CH, CH), jnp.float32),
                jax.ShapeDtypeStruct((nu * D // CH, CH), jnp.float32)),
      mesh=mesh,
      compiler_params=pltpu.CompilerParams(use_tc_tiling_on_sc=True,
                                           needs_layout_passes=False),
      scratch_types=[
          pltpu.VMEM((2 * SPG, CH), jnp.int32),
          pltpu.VMEM((2 * GROUP,), jnp.int32),
          pltpu.VMEM((2 * GROUP,), jnp.int32),
          pltpu.VMEM((2 * GROUP, CH), jnp.float32),
          pltpu.VMEM((OROWS, CH), jnp.float32),
          pltpu.SemaphoreType.DMA,
          pltpu.SemaphoreType.DMA,
          pltpu.SemaphoreType.DMA,
      ],
  )
  return fn(item_idx, user_idx, item_tab, user_tab)


def _project_packed(x4, w, bm4):
  m4 = x4.shape[0]

  def mm(x_ref, w_ref, o_ref):
    x = x_ref[...]
    for o in range(PACK):
      o_ref[:, o * H:(o + 1) * H] = lax.dot_general(
          x[:, o * D:(o + 1) * D], w_ref[...],
          (((1,), (1,)), ((), ())), preferred_element_type=jnp.float32)

  return pl.pallas_call(
      mm,
      grid=(m4 // bm4,),
      in_specs=[pl.BlockSpec((bm4, PACK * D), lambda i: (i, 0)),
                pl.BlockSpec((H, D), lambda i: (0, 0))],
      out_specs=pl.BlockSpec((bm4, PACK * H), lambda i: (i, 0)),
      out_shape=jax.ShapeDtypeStruct((m4, PACK * H), jnp.float32),
  )(x4, w)


def kernel(user_ids, item_ids, user_table, item_table, W_user, W_item):
  b, l = item_ids.shape
  ni = b * l
  item_idx = item_ids.T.reshape(ni // CH, CH)
  user_idx = user_ids.reshape(b // CH, CH)
  t_item = _repack_tc(item_table)
  t_user = _repack_tc(user_table)
  items_f, users_f = _sc_gather(item_idx, user_idx, t_item, t_user, ni, b)
  u_proj = _project_packed(users_f, W_user, 1024).reshape(b, H)
  i_proj = _project_packed(items_f, W_item, 1024).reshape(l, b, H)
  return (u_proj, i_proj.transpose(1, 0, 2))


# speedup vs baseline: 1.3988x; 1.0850x over previous
import jax
import jax.numpy as jnp
from jax import lax
from jax.experimental import pallas as pl
from jax.experimental.pallas import tpu as pltpu
from jax.experimental.pallas import tpu_sc as plsc

D = 32
H = 768
NC = 2
NS = 16
NW = NC * NS
CH = 128
SPG = 2
GROUP = CH * SPG
OROWS = GROUP * D // CH
PACK = CH // D


def _repack_tc(t):
  tt = jnp.transpose(t)
  v = t.shape[0]
  bn = 8192

  def body(x_ref, o_ref):
    xt = x_ref[...].T
    x3 = xt.reshape(bn // PACK, PACK, D)
    o_ref[...] = jnp.concatenate([x3[:, j, :] for j in range(PACK)], axis=1)

  return pl.pallas_call(
      body, grid=(pl.cdiv(v, bn),),
      in_specs=[pl.BlockSpec((D, bn), lambda i: (0, i))],
      out_specs=pl.BlockSpec((bn // PACK, PACK * D), lambda i: (i, 0)),
      out_shape=jax.ShapeDtypeStruct((v // PACK, PACK * D), jnp.float32),
  )(tt)


def _gather_pipe(ids_ref, tab_ref, out_ref, n, wid,
                 ids_v, sidx, cbase, gbuf, obuf,
                 sem_i, sem_g, sem_o):
  per_w = n // NW
  groups = per_w // GROUP
  idrow0 = wid * (per_w // CH)
  orow0 = wid * (per_w * D // CH)

  def fire_ids(g, p):
    return pltpu.async_copy(ids_ref.at[pl.ds(idrow0 + g * SPG, SPG)],
                            ids_v.at[pl.ds(p * SPG, SPG)], sem_i)

  def idx_compute(p):
    for j in range(SPG):
      row = ids_v.at[p * SPG + j]
      for t in range(CH // 16):
        raw = row[pl.ds(t * 16, 16)]
        base = p * GROUP + j * CH + t * 16
        sidx[pl.ds(base, 16)] = lax.shift_right_logical(raw, 2)
        cbase[pl.ds(base, 16)] = (raw & (PACK - 1)) * D

  def fire_gather(p):
    return [pltpu.async_copy(
        tab_ref.at[sidx.at[pl.ds(p * GROUP + j * CH, CH)]],
        gbuf.at[pl.ds(p * GROUP + j * CH, CH)], sem_g)
        for j in range(SPG)]

  def fire_out(g):
    return pltpu.async_copy(obuf, out_ref.at[pl.ds(orow0 + g * OROWS, OROWS)],
                            sem_o)

  def extract(p):
    @pl.loop(0, GROUP // 16)
    def _t(t):
      r0 = pl.multiple_of(t * 16, 16)
      rows = r0 + jnp.arange(16, dtype=jnp.int32)
      cb = cbase[pl.ds(p * GROUP + r0, 16)]
      grows = rows + p * GROUP
      pbase = rows * D
      for c in range(D):
        v = plsc.load_gather(gbuf, [grows, cb + c])
        pp = pbase + c
        plsc.store_scatter(obuf, [lax.shift_right_logical(pp, 7), pp & 127], v)

  ids_t = fire_ids(0, 0)
  ids_t.wait()
  idx_compute(0)
  gat_t = fire_gather(0)
  out_t = None
  if groups > 1:
    ids_t2 = fire_ids(1, 1)

  def group_body(g, p):
    q = 1 - p

    @pl.when(g < groups - 1)
    def _prep():
      ids_t.wait()
      idx_compute(q)

      @pl.when(g < groups - 2)
      def _pre_ids():
        fire_ids(g + 2, p)

      fire_gather(q)

    for cp in gat_t:
      cp.wait()

    @pl.when(g >= 1)
    def _wout():
      nonlocal_out_wait()

    extract(p)
    fire_out(g)

  def nonlocal_out_wait():
    pltpu.make_async_copy(obuf, out_ref.at[pl.ds(orow0, OROWS)],
                          sem_o).wait()

  @pl.loop(0, groups // 2)
  def _k(k):
    group_body(2 * k, 0)
    group_body(2 * k + 1, 1)

  nonlocal_out_wait()


def _sc_gather(idx, tab, n):
  mesh = plsc.VectorSubcoreMesh(core_axis_name="c", subcore_axis_name="s")

  def body(idx_ref, tab_ref, out_ref, ids_v, sidx, cbase, gbuf, obuf,
           sem_i, sem_g, sem_o):
    wid = lax.axis_index("s") * NC + lax.axis_index("c")
    _gather_pipe(idx_ref, tab_ref, out_ref, n, wid,
                 ids_v, sidx, cbase, gbuf, obuf, sem_i, sem_g, sem_o)

  fn = pl.kernel(
      body,
      out_type=jax.ShapeDtypeStruct((n * D // CH, CH), jnp.float32),
      mesh=mesh,
      compiler_params=pltpu.CompilerParams(use_tc_tiling_on_sc=True,
                                           needs_layout_passes=False),
      scratch_types=[
          pltpu.VMEM((2 * SPG, CH), jnp.int32),
          pltpu.VMEM((2 * GROUP,), jnp.int32),
          pltpu.VMEM((2 * GROUP,), jnp.int32),
          pltpu.VMEM((2 * GROUP, CH), jnp.float32),
          pltpu.VMEM((OROWS, CH), jnp.float32),
          pltpu.SemaphoreType.DMA,
          pltpu.SemaphoreType.DMA,
          pltpu.SemaphoreType.DMA,
      ],
  )
  return fn(idx, tab)


def _project_packed(x4, w, bm4):
  m4 = x4.shape[0]

  def mm(x_ref, w_ref, o_ref):
    x = x_ref[...]
    for o in range(PACK):
      o_ref[:, o * H:(o + 1) * H] = lax.dot_general(
          x[:, o * D:(o + 1) * D], w_ref[...],
          (((1,), (1,)), ((), ())), preferred_element_type=jnp.float32)

  return pl.pallas_call(
      mm,
      grid=(m4 // bm4,),
      in_specs=[pl.BlockSpec((bm4, PACK * D), lambda i: (i, 0)),
                pl.BlockSpec((H, D), lambda i: (0, 0))],
      out_specs=pl.BlockSpec((bm4, PACK * H), lambda i: (i, 0)),
      out_shape=jax.ShapeDtypeStruct((m4, PACK * H), jnp.float32),
  )(x4, w)


def kernel(user_ids, item_ids, user_table, item_table, W_user, W_item):
  b, l = item_ids.shape
  ni = b * l
  item_idx = item_ids.T.reshape(ni // CH, CH)
  user_idx = user_ids.reshape(b // CH, CH)
  t_user = _repack_tc(user_table)
  users_f = _sc_gather(user_idx, t_user, b)
  t_item = _repack_tc(item_table)
  items_f = _sc_gather(item_idx, t_item, ni)
  u_proj = _project_packed(users_f, W_user, 1024).reshape(b, H)
  i_proj = _project_packed(items_f, W_item, 1024).reshape(l, b, H)
  return (u_proj, i_proj.transpose(1, 0, 2))


# speedup vs baseline: 1.5144x; 1.0826x over previous
import jax
import jax.numpy as jnp
from jax import lax
from jax.experimental import pallas as pl
from jax.experimental.pallas import tpu as pltpu
from jax.experimental.pallas import tpu_sc as plsc

D = 32
H = 768
NC = 2
NS = 16
NW = NC * NS
CH = 128
SPG = 2
GROUP = CH * SPG
OROWS = GROUP * D // CH
PACK = CH // D


def _repack_tc(t):
  tt = jnp.transpose(t)
  v = t.shape[0]
  bn = 8192

  def body(x_ref, o_ref):
    xt = x_ref[...].T
    x3 = xt.reshape(bn // PACK, PACK, D)
    o_ref[...] = jnp.concatenate([x3[:, j, :] for j in range(PACK)], axis=1)

  return pl.pallas_call(
      body, grid=(pl.cdiv(v, bn),),
      in_specs=[pl.BlockSpec((D, bn), lambda i: (0, i))],
      out_specs=pl.BlockSpec((bn // PACK, PACK * D), lambda i: (i, 0)),
      out_shape=jax.ShapeDtypeStruct((v // PACK, PACK * D), jnp.float32),
  )(tt)


def _gather_pipe(ids_ref, tab_ref, out_ref, n, wid,
                 ids_v, sidx, cbase, gbuf, obuf,
                 sem_i, sem_g, sem_o):
  per_w = n // NW
  groups = per_w // GROUP
  idrow0 = wid * (per_w // CH)
  orow0 = wid * (per_w * D // CH)

  def fire_ids(g, p):
    return pltpu.async_copy(ids_ref.at[pl.ds(idrow0 + g * SPG, SPG)],
                            ids_v.at[pl.ds(p * SPG, SPG)], sem_i)

  def idx_compute(p):
    for j in range(SPG):
      row = ids_v.at[p * SPG + j]
      for t in range(CH // 16):
        raw = row[pl.ds(t * 16, 16)]
        base = p * GROUP + j * CH + t * 16
        sidx[pl.ds(base, 16)] = lax.shift_right_logical(raw, 2)
        cbase[pl.ds(base, 16)] = (raw & (PACK - 1)) * D

  def fire_gather(p):
    return [pltpu.async_copy(
        tab_ref.at[sidx.at[pl.ds(p * GROUP + j * CH, CH)]],
        gbuf.at[pl.ds(p * GROUP + j * CH, CH)], sem_g)
        for j in range(SPG)]

  def fire_out(g):
    return pltpu.async_copy(obuf, out_ref.at[pl.ds(orow0 + g * OROWS, OROWS)],
                            sem_o)

  def extract(p):
    @pl.loop(0, GROUP // 16)
    def _t(t):
      r0 = pl.multiple_of(t * 16, 16)
      rows = r0 + jnp.arange(16, dtype=jnp.int32)
      cb = cbase[pl.ds(p * GROUP + r0, 16)]
      grows = rows + p * GROUP
      pbase = rows * D
      for c in range(D):
        v = plsc.load_gather(gbuf, [grows, cb + c])
        pp = pbase + c
        plsc.store_scatter(obuf, [lax.shift_right_logical(pp, 7), pp & 127], v)

  ids_t = fire_ids(0, 0)
  ids_t.wait()
  idx_compute(0)
  gat_t = fire_gather(0)
  out_t = None
  if groups > 1:
    ids_t2 = fire_ids(1, 1)

  def group_body(g, p):
    q = 1 - p

    @pl.when(g < groups - 1)
    def _prep():
      ids_t.wait()
      idx_compute(q)

      @pl.when(g < groups - 2)
      def _pre_ids():
        fire_ids(g + 2, p)

      fire_gather(q)

    for cp in gat_t:
      cp.wait()

    @pl.when(g >= 1)
    def _wout():
      nonlocal_out_wait()

    extract(p)
    fire_out(g)

  def nonlocal_out_wait():
    pltpu.make_async_copy(obuf, out_ref.at[pl.ds(orow0, OROWS)],
                          sem_o).wait()

  @pl.loop(0, groups // 2)
  def _k(k):
    group_body(2 * k, 0)
    group_body(2 * k + 1, 1)

  nonlocal_out_wait()


def _sc_gather(idx, tab, n):
  mesh = plsc.VectorSubcoreMesh(core_axis_name="c", subcore_axis_name="s")

  def body(idx_ref, tab_ref, out_ref, ids_v, sidx, cbase, gbuf, obuf,
           sem_i, sem_g, sem_o):
    wid = lax.axis_index("s") * NC + lax.axis_index("c")
    _gather_pipe(idx_ref, tab_ref, out_ref, n, wid,
                 ids_v, sidx, cbase, gbuf, obuf, sem_i, sem_g, sem_o)

  fn = pl.kernel(
      body,
      out_type=jax.ShapeDtypeStruct((n * D // CH, CH), jnp.float32),
      mesh=mesh,
      compiler_params=pltpu.CompilerParams(use_tc_tiling_on_sc=True,
                                           needs_layout_passes=False),
      scratch_types=[
          pltpu.VMEM((2 * SPG, CH), jnp.int32),
          pltpu.VMEM((2 * GROUP,), jnp.int32),
          pltpu.VMEM((2 * GROUP,), jnp.int32),
          pltpu.VMEM((2 * GROUP, CH), jnp.float32),
          pltpu.VMEM((OROWS, CH), jnp.float32),
          pltpu.SemaphoreType.DMA,
          pltpu.SemaphoreType.DMA,
          pltpu.SemaphoreType.DMA,
      ],
  )
  return fn(idx, tab)


def _project_packed(x4, w, bm4):
  m4 = x4.shape[0]

  def mm(x_ref, w_ref, o_ref):
    x = x_ref[...]
    for o in range(PACK):
      o_ref[:, o * H:(o + 1) * H] = lax.dot_general(
          x[:, o * D:(o + 1) * D], w_ref[...],
          (((1,), (1,)), ((), ())), preferred_element_type=jnp.float32)

  return pl.pallas_call(
      mm,
      grid=(m4 // bm4,),
      in_specs=[pl.BlockSpec((bm4, PACK * D), lambda i: (i, 0)),
                pl.BlockSpec((H, D), lambda i: (0, 0))],
      out_specs=pl.BlockSpec((bm4, PACK * H), lambda i: (i, 0)),
      out_shape=jax.ShapeDtypeStruct((m4, PACK * H), jnp.float32),
  )(x4, w)


def _project_two(x1, x2, w, bm4):
  m4h = x1.shape[0]
  nb = m4h // bm4

  def mm(x_ref, w_ref, o_ref):
    x = x_ref[...]
    for o in range(PACK):
      o_ref[:, o * H:(o + 1) * H] = lax.dot_general(
          x[:, o * D:(o + 1) * D], w_ref[...],
          (((1,), (1,)), ((), ())), preferred_element_type=jnp.float32)

  buf = pl.pallas_call(
      mm, grid=(nb,),
      in_specs=[pl.BlockSpec((bm4, PACK * D), lambda i: (i, 0)),
                pl.BlockSpec((H, D), lambda i: (0, 0))],
      out_specs=pl.BlockSpec((bm4, PACK * H), lambda i: (i, 0)),
      out_shape=jax.ShapeDtypeStruct((2 * m4h, PACK * H), jnp.float32),
  )(x1, w)

  def mm2(x_ref, w_ref, buf_ref, o_ref):
    mm(x_ref, w_ref, o_ref)

  return pl.pallas_call(
      mm2, grid=(nb,),
      in_specs=[pl.BlockSpec((bm4, PACK * D), lambda i: (i, 0)),
                pl.BlockSpec((H, D), lambda i: (0, 0)),
                pl.BlockSpec(memory_space=pl.ANY)],
      out_specs=pl.BlockSpec((bm4, PACK * H), lambda i: (i + nb, 0)),
      out_shape=jax.ShapeDtypeStruct((2 * m4h, PACK * H), jnp.float32),
      input_output_aliases={2: 0},
  )(x2, w, buf)


def kernel(user_ids, item_ids, user_table, item_table, W_user, W_item):
  b, l = item_ids.shape
  ni = b * l
  item_idx = item_ids.T.reshape(ni // CH, CH)
  user_idx = user_ids.reshape(b // CH, CH)
  t_user = _repack_tc(user_table)
  users_f = _sc_gather(user_idx, t_user, b)
  t_item = _repack_tc(item_table)
  hrows = ni // CH // 2
  items_f1 = _sc_gather(item_idx[:hrows], t_item, ni // 2)
  u_proj = _project_packed(users_f, W_user, 1024).reshape(b, H)
  items_f2 = _sc_gather(item_idx[hrows:], t_item, ni // 2)
  i_flat = _project_two(items_f1, items_f2, W_item, 1024)
  i_proj = i_flat.reshape(l, b, H)
  return (u_proj, i_proj.transpose(1, 0, 2))
